# baseline (speedup 1.0000x reference)
import functools

import jax
import jax.numpy as jnp
from jax import lax
from jax.experimental import pallas as pl
from jax.experimental.pallas import tpu as pltpu
from jax.experimental.pallas import tpu_sc as plsc

_NC, _NS = 2, 16
_NW = _NC * _NS
_SEQ_PER_CHUNK = 2
_EPS = 1e-6


def _vsqrt(x):
    i = plsc.bitcast(x, jnp.int32)
    y = plsc.bitcast(jnp.int32(0x5F3759DF) - (i >> 1), jnp.float32)
    for _ in range(3):
        y = y * (1.5 - 0.5 * x * y * y)
    return x * y


def _vlog(x):
    i = plsc.bitcast(x, jnp.int32)
    e = (i >> 23) - 127
    m = plsc.bitcast((i & jnp.int32(0x007FFFFF)) | jnp.int32(0x3F800000),
                     jnp.float32)
    big = m > 1.4142135
    m = jnp.where(big, m * 0.5, m)
    ef = jnp.where(big, e + 1, e).astype(jnp.float32)
    z = (m - 1.0) / (m + 1.0)
    z2 = z * z
    p = z2 * (0.33333333 + z2 * (0.2 + z2 * (0.14285715 + z2 * 0.11111111)))
    return ef * 0.6931472 + 2.0 * z * (1.0 + p)


def _coeff(s):
    x0 = _vsqrt(jnp.maximum(1.0 + s, _EPS))
    xm = jnp.maximum(x0, 1.0 + _EPS)
    alpha = _vlog(xm + _vsqrt((xm - 1.0) * (xm + 1.0)))
    denom = _vsqrt(jnp.maximum((x0 - 1.0) * (x0 + 1.0), _EPS))
    return alpha / denom


def _scale_seq(buf, base, t, feat, lane):
    starts = list(range(0, t - 15, 16))
    if t % 16:
        starts.append(t - 16)
    nk = feat // 16

    cfs = []
    for o in starts:
        svec = jnp.zeros((16,), jnp.float32)
        for r in range(16):
            row = base + o + r
            acc = None
            for k in range(nk):
                v = buf[row, pl.ds(k * 16, 16)]
                acc = v * v if acc is None else acc + v * v
            svec = jnp.where(lane == r, jnp.sum(acc), svec)
        cfs.append(_coeff(svec))

    done = 0
    for o, cf in zip(starts, cfs):
        for r in range(16):
            if o + r < done:
                continue
            row = base + o + r
            c = cf[r]
            for k in range(nk):
                buf[row, pl.ds(k * 16, 16)] = buf[row, pl.ds(k * 16, 16)] * c
        done = o + 16


@functools.lru_cache(maxsize=None)
def _make_sc_kernel(nb, t, tpad, feat):
    nb_per_w = nb // _NW
    n_chunk = nb_per_w // _SEQ_PER_CHUNK
    rows_per_chunk = 128
    mesh = plsc.VectorSubcoreMesh(core_axis_name="c", subcore_axis_name="s",
                                  num_cores=_NC, num_subcores=_NS)

    @functools.partial(
        pl.kernel,
        out_type=jax.ShapeDtypeStruct((nb * t, feat), jnp.float32),
        mesh=mesh,
        compiler_params=pltpu.CompilerParams(needs_layout_passes=False,
                                             use_tc_tiling_on_sc=False),
        scratch_types=[
            pltpu.VMEM((n_chunk, rows_per_chunk), jnp.int32),
            pltpu.VMEM((rows_per_chunk, feat), jnp.float32),
            pltpu.VMEM((rows_per_chunk, feat), jnp.float32),
            pltpu.SemaphoreType.DMA,
            pltpu.SemaphoreType.DMA,
            pltpu.SemaphoreType.DMA,
            pltpu.SemaphoreType.DMA,
        ],
    )
    def lorentz_sc(ids_hbm, tab_hbm, out_hbm,
                   idxb, buf0, buf1, gsem0, gsem1, ssem0, ssem1):
        wid = lax.axis_index("s") * _NC + lax.axis_index("c")
        base = wid * nb_per_w
        lane = lax.broadcasted_iota(jnp.int32, (16,), 0)
        pltpu.sync_copy(ids_hbm.at[wid], idxb)

        bufs = (buf0, buf1)
        gsems = (gsem0, gsem1)
        ssems = (ssem0, ssem1)

        def gather(c, b):
            return pltpu.make_async_copy(tab_hbm.at[idxb.at[c]],
                                         bufs[b], gsems[b])

        def store(c, s, b):
            seq = base + c * _SEQ_PER_CHUNK + s
            return pltpu.make_async_copy(
                bufs[b].at[pl.ds(s * tpad, t)],
                out_hbm.at[pl.ds(seq * t, t)], ssems[b])

        def compute(b):
            def seq(s, carry):
                _scale_seq(bufs[b], s * tpad, t, feat, lane)
                return carry
            lax.fori_loop(0, _SEQ_PER_CHUNK, seq, 0)

        def fire_stores(c, b):
            for s in range(_SEQ_PER_CHUNK):
                store(c, s, b).start()

        def drain_stores(c, b):
            for s in range(_SEQ_PER_CHUNK):
                store(c, s, b).wait()

        gather(0, 0).start()

        def step(i, carry):
            c0 = 2 * i
            c1 = c0 + 1
            with jax.named_scope("p_gwait0"):
                gather(c0, 0).wait()

            with jax.named_scope("p_drain1"):
                @pl.when(i > 0)
                def _():
                    drain_stores(c1 - 2, 1)

            with jax.named_scope("p_gstart1"):
                gather(c1, 1).start()
            with jax.named_scope("p_comp0"):
                compute(0)
            with jax.named_scope("p_fire0"):
                fire_stores(c0, 0)
            with jax.named_scope("p_gwait1"):
                gather(c1, 1).wait()

            with jax.named_scope("p_drain0"):
                @pl.when(i < n_chunk // 2 - 1)
                def _():
                    drain_stores(c0, 0)
                    gather(c0 + 2, 0).start()

            with jax.named_scope("p_comp1"):
                compute(1)
            with jax.named_scope("p_fire1"):
                fire_stores(c1, 1)
            return carry

        lax.fori_loop(0, n_chunk // 2, step, 0)
        drain_stores(n_chunk - 2, 0)
        drain_stores(n_chunk - 1, 1)

    return lorentz_sc


def kernel(input_ids, embedding):
    nb, t = input_ids.shape
    feat = embedding.shape[1]
    tpad = -(-t // 8) * 8
    nb_per_w = nb // _NW
    n_chunk = nb_per_w // _SEQ_PER_CHUNK
    ids = jnp.pad(input_ids.astype(jnp.int32), ((0, 0), (0, tpad - t)))
    ids = jnp.reshape(ids, (nb // _SEQ_PER_CHUNK, _SEQ_PER_CHUNK * tpad))
    ids = jnp.pad(ids, ((0, 0), (0, 128 - _SEQ_PER_CHUNK * tpad)))
    ids = jnp.reshape(ids, (_NW, n_chunk, 128))
    out = _make_sc_kernel(nb, t, tpad, feat)(ids, embedding)
    return jnp.reshape(out, (nb, t, feat))


# speedup vs baseline: 2.3692x; 2.3692x over previous
import functools

import jax
import jax.numpy as jnp
from jax import lax
from jax.experimental import pallas as pl
from jax.experimental.pallas import tpu as pltpu
from jax.experimental.pallas import tpu_sc as plsc

_NC, _NS = 2, 16
_NW = _NC * _NS
_F = 64
_CH = 128
_EPS = 1e-6


def _vsqrt(x):
    i = plsc.bitcast(x, jnp.int32)
    y = plsc.bitcast(jnp.int32(0x5F3759DF) - (i >> 1), jnp.float32)
    for _ in range(3):
        y = y * (1.5 - 0.5 * x * y * y)
    return x * y


def _vlog(x):
    i = plsc.bitcast(x, jnp.int32)
    e = (i >> 23) - 127
    m = plsc.bitcast((i & jnp.int32(0x007FFFFF)) | jnp.int32(0x3F800000),
                     jnp.float32)
    big = m > 1.4142135
    m = jnp.where(big, m * 0.5, m)
    ef = jnp.where(big, e + 1, e).astype(jnp.float32)
    z = (m - 1.0) / (m + 1.0)
    z2 = z * z
    p = z2 * (0.33333333 + z2 * (0.2 + z2 * (0.14285715 + z2 * 0.11111111)))
    return ef * 0.6931472 + 2.0 * z * (1.0 + p)


def _coeff(s):
    x0 = _vsqrt(jnp.maximum(1.0 + s, _EPS))
    xm = jnp.maximum(x0, 1.0 + _EPS)
    alpha = _vlog(xm + _vsqrt((xm - 1.0) * (xm + 1.0)))
    denom = _vsqrt(jnp.maximum((x0 - 1.0) * (x0 + 1.0), _EPS))
    return alpha / denom


def _scale_chunk(buf):
    lane = lax.broadcasted_iota(jnp.int32, (16,), 0)

    def group(g, carry):
        svec = jnp.zeros((16,), jnp.float32)
        for r in range(16):
            row = g * 16 + r
            acc = None
            for k in range(_F // 16):
                v = buf[row, pl.ds(k * 16, 16)]
                acc = v * v if acc is None else acc + v * v
            svec = jnp.where(lane == r, jnp.sum(acc), svec)
        cf = _coeff(svec)
        for r in range(16):
            row = g * 16 + r
            c = cf[r]
            for k in range(_F // 16):
                buf[row, pl.ds(k * 16, 16)] = buf[row, pl.ds(k * 16, 16)] * c
        return carry
    lax.fori_loop(0, _CH // 16, group, 0)


@functools.lru_cache(maxsize=None)
def _make_sc_kernel(n_chunk):
    rows_per_w = n_chunk * _CH
    mesh = plsc.VectorSubcoreMesh(core_axis_name="c", subcore_axis_name="s",
                                  num_cores=_NC, num_subcores=_NS)

    @functools.partial(
        pl.kernel,
        out_type=jax.ShapeDtypeStruct((_NW * rows_per_w, _F), jnp.float32),
        mesh=mesh,
        compiler_params=pltpu.CompilerParams(needs_layout_passes=False,
                                             use_tc_tiling_on_sc=False),
        scratch_types=[
            pltpu.VMEM((n_chunk, _CH), jnp.int32),
            pltpu.VMEM((_CH, _F), jnp.float32),
            pltpu.VMEM((_CH, _F), jnp.float32),
            pltpu.SemaphoreType.DMA,
            pltpu.SemaphoreType.DMA,
            pltpu.SemaphoreType.DMA,
            pltpu.SemaphoreType.DMA,
        ],
    )
    def lorentz_sc(ids_hbm, tab_hbm, out_hbm,
                   idxb, buf0, buf1, gsem0, gsem1, ssem0, ssem1):
        wid = lax.axis_index("s") * _NC + lax.axis_index("c")
        base = wid * rows_per_w
        pltpu.sync_copy(ids_hbm.at[wid], idxb)

        bufs = (buf0, buf1)
        gsems = (gsem0, gsem1)
        ssems = (ssem0, ssem1)

        def gather(c, b):
            return pltpu.make_async_copy(tab_hbm.at[idxb.at[c]],
                                         bufs[b], gsems[b])

        def store(c, b):
            return pltpu.make_async_copy(
                bufs[b], out_hbm.at[pl.ds(base + c * _CH, _CH)], ssems[b])

        gather(0, 0).start()

        def step(i, carry):
            c0 = 2 * i
            c1 = c0 + 1
            gather(c0, 0).wait()

            @pl.when(i > 0)
            def _():
                store(c1 - 2, 1).wait()

            gather(c1, 1).start()
            _scale_chunk(buf0)
            store(c0, 0).start()
            gather(c1, 1).wait()

            @pl.when(i < n_chunk // 2 - 1)
            def _():
                store(c0, 0).wait()
                gather(c0 + 2, 0).start()

            _scale_chunk(buf1)
            store(c1, 1).start()
            return carry

        lax.fori_loop(0, n_chunk // 2, step, 0)
        store(n_chunk - 2, 0).wait()
        store(n_chunk - 1, 1).wait()

    return lorentz_sc


def kernel(input_ids, embedding):
    b, t = input_ids.shape
    total = b * t
    rows_per_w = total // _NW
    n_chunk = rows_per_w // _CH
    ids = jnp.reshape(input_ids.astype(jnp.int32), (_NW, n_chunk, _CH))
    out = _make_sc_kernel(n_chunk)(ids, embedding)
    return jnp.reshape(out, (b, t, _F))


# speedup vs baseline: 2.3874x; 1.0077x over previous
import functools

import jax
import jax.numpy as jnp
from jax import lax
from jax.experimental import pallas as pl
from jax.experimental.pallas import tpu as pltpu
from jax.experimental.pallas import tpu_sc as plsc

_NC, _NS = 2, 16
_NW = _NC * _NS
_SEQ_PER_CHUNK = 2
_EPS = 1e-6


def _vsqrt(x):
    i = plsc.bitcast(x, jnp.int32)
    y = plsc.bitcast(jnp.int32(0x5F3759DF) - (i >> 1), jnp.float32)
    for _ in range(3):
        y = y * (1.5 - 0.5 * x * y * y)
    return x * y


def _vlog(x):
    i = plsc.bitcast(x, jnp.int32)
    e = (i >> 23) - 127
    m = plsc.bitcast((i & jnp.int32(0x007FFFFF)) | jnp.int32(0x3F800000),
                     jnp.float32)
    big = m > 1.4142135
    m = jnp.where(big, m * 0.5, m)
    ef = jnp.where(big, e + 1, e).astype(jnp.float32)
    z = (m - 1.0) / (m + 1.0)
    z2 = z * z
    p = z2 * (0.33333333 + z2 * (0.2 + z2 * (0.14285715 + z2 * 0.11111111)))
    return ef * 0.6931472 + 2.0 * z * (1.0 + p)


def _coeff(s):
    x0 = _vsqrt(jnp.maximum(1.0 + s, _EPS))
    xm = jnp.maximum(x0, 1.0 + _EPS)
    alpha = _vlog(xm + _vsqrt((xm - 1.0) * (xm + 1.0)))
    denom = _vsqrt(jnp.maximum((x0 - 1.0) * (x0 + 1.0), _EPS))
    return alpha / denom


def _scale_seq(buf, base, t, feat, lane):
    starts = list(range(0, t - 15, 16))
    if t % 16:
        starts.append(t - 16)
    nk = feat // 16

    cfs = []
    for o in starts:
        svec = jnp.zeros((16,), jnp.float32)
        for r in range(16):
            row = base + o + r
            acc = None
            for k in range(nk):
                v = buf[row, pl.ds(k * 16, 16)]
                acc = v * v if acc is None else acc + v * v
            svec = jnp.where(lane == r, jnp.sum(acc), svec)
        cfs.append(_coeff(svec))

    done = 0
    for o, cf in zip(starts, cfs):
        for r in range(16):
            if o + r < done:
                continue
            row = base + o + r
            c = cf[r]
            for k in range(nk):
                buf[row, pl.ds(k * 16, 16)] = buf[row, pl.ds(k * 16, 16)] * c
        done = o + 16


@functools.lru_cache(maxsize=None)
def _make_sc_kernel(nb, t, tpad, feat):
    nb_per_w = nb // _NW
    n_chunk = nb_per_w // _SEQ_PER_CHUNK
    rows_per_chunk = 128
    mesh = plsc.VectorSubcoreMesh(core_axis_name="c", subcore_axis_name="s",
                                  num_cores=_NC, num_subcores=_NS)

    @functools.partial(
        pl.kernel,
        out_type=jax.ShapeDtypeStruct((nb, t, feat), jnp.float32),
        mesh=mesh,
        compiler_params=pltpu.CompilerParams(needs_layout_passes=False,
                                             use_tc_tiling_on_sc=False),
        scratch_types=[
            pltpu.VMEM((n_chunk, rows_per_chunk), jnp.int32),
            pltpu.VMEM((rows_per_chunk, feat), jnp.float32),
            pltpu.VMEM((rows_per_chunk, feat), jnp.float32),
            pltpu.SemaphoreType.DMA,
            pltpu.SemaphoreType.DMA,
            pltpu.SemaphoreType.DMA,
            pltpu.SemaphoreType.DMA,
        ],
    )
    def lorentz_sc(ids_hbm, tab_hbm, out_hbm,
                   idxb, buf0, buf1, gsem0, gsem1, ssem0, ssem1):
        wid = lax.axis_index("s") * _NC + lax.axis_index("c")
        base = wid * nb_per_w
        lane = lax.broadcasted_iota(jnp.int32, (16,), 0)
        pltpu.sync_copy(ids_hbm.at[wid], idxb)

        bufs = (buf0, buf1)
        gsems = (gsem0, gsem1)
        ssems = (ssem0, ssem1)

        def gather(c, b):
            return pltpu.make_async_copy(tab_hbm.at[idxb.at[c]],
                                         bufs[b], gsems[b])

        def store(c, s, b):
            return pltpu.make_async_copy(
                bufs[b].at[pl.ds(s * tpad, t)],
                out_hbm.at[base + c * _SEQ_PER_CHUNK + s], ssems[b])

        def compute(b):
            def seq(s, carry):
                _scale_seq(bufs[b], s * tpad, t, feat, lane)
                return carry
            lax.fori_loop(0, _SEQ_PER_CHUNK, seq, 0)

        def fire_stores(c, b):
            for s in range(_SEQ_PER_CHUNK):
                store(c, s, b).start()

        def drain_stores(c, b):
            for s in range(_SEQ_PER_CHUNK):
                store(c, s, b).wait()

        gather(0, 0).start()

        def step(i, carry):
            c0 = 2 * i
            c1 = c0 + 1
            with jax.named_scope("p_gwait0"):
                gather(c0, 0).wait()

            with jax.named_scope("p_drain1"):
                @pl.when(i > 0)
                def _():
                    drain_stores(c1 - 2, 1)

            with jax.named_scope("p_gstart1"):
                gather(c1, 1).start()
            with jax.named_scope("p_comp0"):
                compute(0)
            with jax.named_scope("p_fire0"):
                fire_stores(c0, 0)
            with jax.named_scope("p_gwait1"):
                gather(c1, 1).wait()

            with jax.named_scope("p_drain0"):
                @pl.when(i < n_chunk // 2 - 1)
                def _():
                    drain_stores(c0, 0)
                    gather(c0 + 2, 0).start()

            with jax.named_scope("p_comp1"):
                compute(1)
            with jax.named_scope("p_fire1"):
                fire_stores(c1, 1)
            return carry

        lax.fori_loop(0, n_chunk // 2, step, 0)
        drain_stores(n_chunk - 2, 0)
        drain_stores(n_chunk - 1, 1)

    return lorentz_sc


def _spread_pad(x, npad, nrows):
    n = x.shape[0]
    base = jnp.arange(n, dtype=jnp.int32)[:, None] * npad
    pad = (base + jnp.arange(npad, dtype=jnp.int32)[None, :]) % nrows
    return jnp.concatenate([x, pad], axis=1)


def kernel(input_ids, embedding):
    nb, t = input_ids.shape
    nrows, feat = embedding.shape
    tpad = -(-t // 8) * 8
    nb_per_w = nb // _NW
    n_chunk = nb_per_w // _SEQ_PER_CHUNK
    ids = _spread_pad(input_ids.astype(jnp.int32), tpad - t, nrows)
    ids = jnp.reshape(ids, (nb // _SEQ_PER_CHUNK, _SEQ_PER_CHUNK * tpad))
    ids = _spread_pad(ids, 128 - _SEQ_PER_CHUNK * tpad, nrows)
    ids = jnp.reshape(ids, (_NW, n_chunk, 128))
    return _make_sc_kernel(nb, t, tpad, feat)(ids, embedding)
